# Initial kernel scaffold; baseline (speedup 1.0000x reference)
import jax
import jax.numpy as jnp
from jax.experimental import pallas as pl


def kernel(x, batch, W_d1, b_d1, W_e1, b_e1, W_d2, b_d2, W_e2, b_e2, W_l1, b_l1, W_m1, b_m1, W_m2, b_m2, W_m3, b_m3):
    raise NotImplementedError("write your pallas kernel here")


# speedup vs baseline: 9.5682x; 9.5682x over previous
import functools

import jax
import jax.numpy as jnp
from jax import lax
from jax.experimental import pallas as pl
from jax.experimental.pallas import tpu as pltpu

_B = 8
_M = 1024
_K = 20
_F = 64

_BIG = 3e38
_SELF = 1e10


def _edge_body(xr_ref, xc_ref, A_ref, Wj_ref, c_ref, out_ref, *, R):
    rb = pl.program_id(1)
    xr = xr_ref[0]
    xc = xc_ref[0]
    d = xr.shape[1]
    ones_r = jnp.ones((1, d), jnp.float32)
    sq_r = lax.dot_general(xr * xr, ones_r, (((1,), (1,)), ((), ())),
                           preferred_element_type=jnp.float32)
    sq_c = lax.dot_general(ones_r, xc * xc, (((1,), (1,)), ((), ())),
                           preferred_element_type=jnp.float32)
    inner = lax.dot_general(xr, xc, (((1,), (1,)), ((), ())),
                            preferred_element_type=jnp.float32)
    dist = sq_r - 2.0 * inner + sq_c
    row_g = rb * R + lax.broadcasted_iota(jnp.int32, (R, _M), 0)
    col = lax.broadcasted_iota(jnp.int32, (R, _M), 1)
    dist = jnp.where(col == row_g, jnp.float32(_SELF), dist)

    p = jnp.dot(xc, Wj_ref[...], preferred_element_type=jnp.float32)

    removed = jnp.zeros((R, _M), jnp.bool_)
    accmax = jnp.full((R, _F), -_BIG, jnp.float32)
    accmin = jnp.full((R, _F), _BIG, jnp.float32)
    for _ in range(_K):
        cur = jnp.where(removed, jnp.float32(_BIG), dist)
        m = jnp.min(cur, axis=1, keepdims=True)
        ismin = cur == m
        jsel = jnp.min(jnp.where(ismin, col, jnp.int32(1 << 30)), axis=1,
                       keepdims=True)
        newly = col == jsel
        removed = removed | newly
        g = jnp.dot(newly.astype(jnp.float32), p,
                    preferred_element_type=jnp.float32)
        accmax = jnp.maximum(accmax, g[:, :_F])
        accmin = jnp.minimum(accmin, g[:, _F:])

    out = (jnp.dot(xr, A_ref[...], preferred_element_type=jnp.float32)
           + c_ref[...] + accmax - accmin)
    out_ref[0] = out


def _edge_layer(xb, A, Wj, c, R=256):
    d = xb.shape[-1]
    grid = (_B, _M // R)
    return pl.pallas_call(
        functools.partial(_edge_body, R=R),
        grid=grid,
        in_specs=[
            pl.BlockSpec((1, R, d), lambda b, r: (b, r, 0)),
            pl.BlockSpec((1, _M, d), lambda b, r: (b, 0, 0)),
            pl.BlockSpec((d, _F), lambda b, r: (0, 0)),
            pl.BlockSpec((d, 2 * _F), lambda b, r: (0, 0)),
            pl.BlockSpec((1, _F), lambda b, r: (0, 0)),
        ],
        out_specs=pl.BlockSpec((1, R, _F), lambda b, r: (b, r, 0)),
        out_shape=jax.ShapeDtypeStruct((_B, _M, _F), jnp.float32),
        compiler_params=pltpu.CompilerParams(
            dimension_semantics=("parallel", "arbitrary")),
    )(xb, xb, A, Wj, c)


def _mlp_body(f_ref, w1_ref, b1_ref, w2_ref, b2_ref, w3_ref, b3_ref,
              w4_ref, b4_ref, out_ref):
    h = jnp.maximum(jnp.dot(f_ref[...], w1_ref[...],
                            preferred_element_type=jnp.float32)
                    + b1_ref[...], 0.0)
    h = jnp.maximum(jnp.dot(h, w2_ref[...],
                            preferred_element_type=jnp.float32)
                    + b2_ref[...], 0.0)
    h = jnp.maximum(jnp.dot(h, w3_ref[...],
                            preferred_element_type=jnp.float32)
                    + b3_ref[...], 0.0)
    z = jnp.dot(h, w4_ref[...], preferred_element_type=jnp.float32) + b4_ref[...]
    zm = jnp.max(z, axis=1, keepdims=True)
    zs = z - zm
    out_ref[...] = zs - jnp.log(jnp.sum(jnp.exp(zs), axis=1, keepdims=True))


def _mlp(feat, W_l1, b_l1, W_m1, b_m1, W_m2, b_m2, W_m3, b_m3, R=1024):
    n = feat.shape[0]
    nc = W_m3.shape[1]
    grid = (n // R,)
    full = lambda a, b: pl.BlockSpec(a, b)
    return pl.pallas_call(
        _mlp_body,
        grid=grid,
        in_specs=[
            pl.BlockSpec((R, feat.shape[1]), lambda i: (i, 0)),
            pl.BlockSpec(W_l1.shape, lambda i: (0, 0)),
            pl.BlockSpec((1, b_l1.shape[0]), lambda i: (0, 0)),
            pl.BlockSpec(W_m1.shape, lambda i: (0, 0)),
            pl.BlockSpec((1, b_m1.shape[0]), lambda i: (0, 0)),
            pl.BlockSpec(W_m2.shape, lambda i: (0, 0)),
            pl.BlockSpec((1, b_m2.shape[0]), lambda i: (0, 0)),
            pl.BlockSpec(W_m3.shape, lambda i: (0, 0)),
            pl.BlockSpec((1, b_m3.shape[0]), lambda i: (0, 0)),
        ],
        out_specs=pl.BlockSpec((R, nc), lambda i: (i, 0)),
        out_shape=jax.ShapeDtypeStruct((n, nc), jnp.float32),
        compiler_params=pltpu.CompilerParams(
            dimension_semantics=("parallel",)),
    )(feat, W_l1, b_l1[None, :], W_m1, b_m1[None, :], W_m2, b_m2[None, :],
      W_m3, b_m3[None, :])


def _prep(Wd, bd, We, be, d, pad_to=None):
    Wd_i, Wd_j = Wd[:d], Wd[d:]
    We_i, We_j = We[:d], We[d:]
    A = (Wd_i - Wd_j) - (We_i - We_j)
    Wj = jnp.concatenate([Wd_j, We_j], axis=1)
    c = (bd - be)[None, :]
    if pad_to is not None and pad_to > d:
        A = jnp.pad(A, ((0, pad_to - d), (0, 0)))
        Wj = jnp.pad(Wj, ((0, pad_to - d), (0, 0)))
    return A, Wj, c


def kernel(x, batch, W_d1, b_d1, W_e1, b_e1, W_d2, b_d2, W_e2, b_e2,
           W_l1, b_l1, W_m1, b_m1, W_m2, b_m2, W_m3, b_m3):
    xb = x.reshape(_B, _M, 3)
    xb8 = jnp.pad(xb, ((0, 0), (0, 0), (0, 5)))
    A1, Wj1, c1 = _prep(W_d1, b_d1, W_e1, b_e1, 3, pad_to=8)
    x1 = _edge_layer(xb8, A1, Wj1, c1)
    A2, Wj2, c2 = _prep(W_d2, b_d2, W_e2, b_e2, 64)
    x2 = _edge_layer(x1, A2, Wj2, c2)
    x3 = _edge_layer(x2, A2, Wj2, c2)
    feat = jnp.concatenate([x1, x2, x3], axis=-1).reshape(_B * _M, 3 * _F)
    return _mlp(feat, W_l1, b_l1, W_m1, b_m1, W_m2, b_m2, W_m3, b_m3)
